# baseline (speedup 1.0000x reference)
import functools

import numpy as np
import jax
import jax.numpy as jnp
from jax import lax
from jax.experimental import pallas as pl
from jax.experimental.pallas import tpu as pltpu
from jax.experimental.pallas import tpu_sc as plsc

N = 10000
E = 320000
D_IN = 128
D_OUT = 128
NB = 32
TD = 2 * NB
NL = 4
HID = 128
K = 6
T = 1.0

NC = 2
NS = 16
NW = NC * NS
B = 128
M = 2
ZB = 128
EPW = 10112
NCH = EPW // B
RNDS = NCH // M
EP = EPW * NW
ACC_ROWS = 10240
RPT = ACC_ROWS // NS
WPT = N // NS

BLK = 2000
GRID = N // BLK

_F32 = jnp.float32


def _consts():
    exp = np.zeros((NB, TD), np.float32)
    p = np.zeros((TD, TD), np.float32)
    for j in range(NB):
        exp[j, 2 * j] = 1.0
        exp[j, 2 * j + 1] = 1.0
        p[2 * j + 1, 2 * j] = -1.0
        p[2 * j, 2 * j + 1] = 1.0
    return jnp.asarray(exp), jnp.asarray(p)


def _sc_agg_body(table_hbm, srcw_hbm, dstw_hbm, out_hbm,
                 acc_sh, idx_s, idx_d, zb_v, *ring):
    rows = ring[:M]
    gsem = ring[M:2 * M]
    ssem = ring[2 * M:3 * M]
    c = lax.axis_index("c")
    s = lax.axis_index("s")
    wid = c * NS + s

    def zfill(i, _):
        r = i // (TD // 16)
        q = (i % (TD // 16)) * 16
        zb_v[r, pl.ds(q, 16)] = jnp.zeros((16,), _F32)
        return 0
    lax.fori_loop(0, ZB * TD // 16, zfill, 0)

    def zacc(i, _):
        pltpu.sync_copy(zb_v, acc_sh.at[pl.ds(s * RPT + i * ZB, ZB)])
        return 0
    lax.fori_loop(0, RPT // ZB, zacc, 0)

    pltpu.sync_copy(srcw_hbm.at[wid], idx_s)
    pltpu.sync_copy(dstw_hbm.at[wid], idx_d)

    pltpu.async_copy(table_hbm.at[idx_s.at[0]], rows[0], gsem[0])

    plsc.subcore_barrier()

    pltpu.make_async_copy(table_hbm.at[idx_s.at[0]], rows[0], gsem[0]).wait()
    pltpu.async_copy(rows[0], acc_sh.at[idx_d.at[0]], ssem[0], add=True)
    pltpu.async_copy(table_hbm.at[idx_s.at[1]], rows[1], gsem[1])

    def step(r, _):
        for q in range(2):
            j = 1 + 2 * r + q
            b = (1 + q) % 2
            jn = jnp.minimum(j + 1, NCH - 1)
            pltpu.make_async_copy(table_hbm.at[idx_s.at[0]], rows[b],
                                  gsem[b]).wait()
            pltpu.async_copy(rows[b], acc_sh.at[idx_d.at[j]], ssem[b],
                             add=True)
            pltpu.make_async_copy(rows[1 - b], acc_sh.at[idx_d.at[0]],
                                  ssem[1 - b]).wait()
            pltpu.async_copy(table_hbm.at[idx_s.at[jn]], rows[1 - b],
                             gsem[1 - b])
        return 0
    lax.fori_loop(0, (NCH - 1) // 2, step, 0)
    pltpu.make_async_copy(table_hbm.at[idx_s.at[0]], rows[1], gsem[1]).wait()
    pltpu.make_async_copy(rows[0], acc_sh.at[idx_d.at[0]], ssem[0]).wait()

    plsc.subcore_barrier()

    pltpu.sync_copy(acc_sh.at[pl.ds(s * RPT, RPT)],
                    out_hbm.at[c, pl.ds(s * RPT, RPT)])


_sc_agg = pl.kernel(
    _sc_agg_body,
    out_type=jax.ShapeDtypeStruct((NC, ACC_ROWS, TD), _F32),
    mesh=plsc.VectorSubcoreMesh(core_axis_name="c", subcore_axis_name="s",
                                num_cores=NC, num_subcores=NS),
    scratch_types=[
        pltpu.VMEM_SHARED((ACC_ROWS, TD), _F32),
        pltpu.VMEM((NCH, B), jnp.int32),
        pltpu.VMEM((NCH, B), jnp.int32),
        pltpu.VMEM((ZB, TD), _F32),
    ] + [pltpu.VMEM((B, TD), _F32) for _ in range(M)]
      + [pltpu.SemaphoreType.DMA for _ in range(2 * M)],
    compiler_params=pltpu.CompilerParams(use_tc_tiling_on_sc=False,
                                         disable_bounds_checks=True),
)


def _gelu(v):
    return 0.5 * v * (1.0 + lax.erf(v * np.float32(0.7071067811865476)))


def _dot(a, b):
    return jnp.dot(a, b, preferred_element_type=_F32)


def _prep_body(x_r, inWT_r, inb_r, dga_r, dgb_r, h_r, dinv_r, d2_r, dsq_r):
    deg = jnp.maximum(dga_r[0][:, :1] + dgb_r[0][:, :1], 1.0)
    dinv = lax.rsqrt(deg)
    dinv_r[...] = jnp.broadcast_to(dinv, (BLK, TD))
    d2_r[...] = jnp.broadcast_to(dinv * dinv, (BLK, TD))
    dsq_r[...] = jnp.broadcast_to(jnp.sqrt(deg), (BLK, TD))
    h_r[...] = _dot(x_r[...], inWT_r[...]) + inb_r[...]


def _phi_body(h_r, dinv_r, w1_r, b1_r, w2_r, b2_r, w3_r, b3_r, lt_r, ltb_r,
              exp_r, p_r, u_r, rs_r, c_r, s_r):
    h = h_r[...]
    a = _gelu(_dot(h, w1_r[...]) + b1_r[...])
    a = _gelu(_dot(a, w2_r[...]) + b2_r[...])
    ang = _dot(a, w3_r[...]) + b3_r[...]
    C = _dot(jnp.cos(ang), exp_r[...])
    S = _dot(jnp.sin(ang), exp_r[...])
    hb = C * h + S * _dot(h, p_r[...])
    Hm = _dot(hb, lt_r[...]) + ltb_r[...]
    u = dinv_r[...] * Hm
    u_r[...] = u
    rs_r[...] = u
    c_r[...] = C
    s_r[...] = S


def _step_body(u_r, d2_r, ga_r, gb_r, rs_r, un_r, rsn_r, *, coef):
    un = np.float32(coef) * (u_r[...] - d2_r[...] * (ga_r[0] + gb_r[0]))
    un_r[...] = un
    rsn_r[...] = rs_r[...] + un


def _tail_body(h_r, rs_r, dsq_r, c_r, s_r, p_r, hn_r):
    res = rs_r[...] * dsq_r[...]
    hout = c_r[...] * res - s_r[...] * _dot(res, p_r[...])
    hn_r[...] = h_r[...] + _gelu(hout)


def _out_body(h_r, wT_r, b_r, o_r):
    o_r[...] = _dot(h_r[...], wT_r[...]) + b_r[...]


def _rows(d):
    return pl.BlockSpec((BLK, d), lambda i: (i, 0))


def _aggspec(c):
    return pl.BlockSpec((1, BLK, TD), lambda i, c=c: (c, i, 0))


def _full(r, d):
    return pl.BlockSpec((r, d), lambda i: (0, 0))


def _sds(*shapes):
    return [jax.ShapeDtypeStruct(s, _F32) for s in shapes]


_prep = pl.pallas_call(
    _prep_body, grid=(GRID,),
    in_specs=[_rows(D_IN), _full(D_IN, TD), _full(1, TD), _aggspec(0), _aggspec(1)],
    out_specs=[_rows(TD)] * 4,
    out_shape=_sds((N, TD), (N, TD), (N, TD), (N, TD)),
)

_phi = pl.pallas_call(
    _phi_body, grid=(GRID,),
    in_specs=[_rows(TD), _rows(TD), _full(TD, HID), _full(1, HID),
              _full(HID, HID), _full(1, HID), _full(HID, NB), _full(1, NB),
              _full(TD, TD), _full(1, TD), _full(NB, TD), _full(TD, TD)],
    out_specs=[_rows(TD)] * 4,
    out_shape=_sds((N, TD), (N, TD), (N, TD), (N, TD)),
)


def _make_step(coef):
    return pl.pallas_call(
        functools.partial(_step_body, coef=coef), grid=(GRID,),
        in_specs=[_rows(TD), _rows(TD), _aggspec(0), _aggspec(1), _rows(TD)],
        out_specs=[_rows(TD)] * 2,
        out_shape=_sds((N, TD), (N, TD)),
    )


_steps = [_make_step(-T / k) for k in range(1, K + 1)]

_tail = pl.pallas_call(
    _tail_body, grid=(GRID,),
    in_specs=[_rows(TD)] * 5 + [_full(TD, TD)],
    out_specs=_rows(TD),
    out_shape=jax.ShapeDtypeStruct((N, TD), _F32),
)

_out = pl.pallas_call(
    _out_body, grid=(GRID,),
    in_specs=[_rows(TD), _full(TD, D_OUT), _full(1, D_OUT)],
    out_specs=_rows(D_OUT),
    out_shape=jax.ShapeDtypeStruct((N, D_OUT), _F32),
)


def kernel(x, edge_index, in_W, in_b, phi_W1, phi_b1, phi_W2, phi_b2,
           phi_W3, phi_b3, lt_W, lt_b, out_W, out_b):
    exp_m, p_m = _consts()
    src = edge_index[0]
    dst = edge_index[1]
    pad = EP - E
    src_w = jnp.concatenate([src, jnp.zeros((pad,), jnp.int32)]).reshape(NW, NCH, B)
    dst_w = jnp.concatenate([dst, jnp.full((pad,), N, jnp.int32)]).reshape(NW, NCH, B)

    dagg = _sc_agg(jnp.ones((N, TD), _F32), src_w, dst_w)
    h, dinv, d2, dsq = _prep(x, in_W.T, in_b.reshape(1, TD), dagg, dagg)

    for l in range(NL):
        u, rs, C, S = _phi(h, dinv,
                           phi_W1[l].T, phi_b1[l].reshape(1, HID),
                           phi_W2[l].T, phi_b2[l].reshape(1, HID),
                           phi_W3[l].T, phi_b3[l].reshape(1, NB),
                           lt_W[l].T, lt_b[l].reshape(1, TD),
                           exp_m, p_m)
        for k in range(K):
            agg = _sc_agg(u, src_w, dst_w)
            u, rs = _steps[k](u, d2, agg, agg, rs)
        h = _tail(h, rs, dsq, C, S, p_m)

    return _out(h, out_W.T, out_b.reshape(1, D_OUT))


# speedup vs baseline: 1.1809x; 1.1809x over previous
import functools

import numpy as np
import jax
import jax.numpy as jnp
from jax import lax
from jax.experimental import pallas as pl
from jax.experimental.pallas import tpu as pltpu
from jax.experimental.pallas import tpu_sc as plsc

N = 10000
E = 320000
D_IN = 128
D_OUT = 128
NB = 32
TD = 2 * NB
NL = 4
HID = 128
K = 6
T = 1.0

NC = 2
NS = 16
NW = NC * NS
B = 128
M = 3
ZB = 128
EPW = 10112
NCH = EPW // B
RNDS = NCH // M
EP = EPW * NW
ACC_ROWS = 10240
RPT = ACC_ROWS // NS
WPT = N // NS

BLK = 2000
GRID = N // BLK

_F32 = jnp.float32


def _consts():
    exp = np.zeros((NB, TD), np.float32)
    p = np.zeros((TD, TD), np.float32)
    for j in range(NB):
        exp[j, 2 * j] = 1.0
        exp[j, 2 * j + 1] = 1.0
        p[2 * j + 1, 2 * j] = -1.0
        p[2 * j, 2 * j + 1] = 1.0
    return jnp.asarray(exp), jnp.asarray(p)


def _sc_agg_body(table_hbm, srcw_hbm, dstw_hbm, out_hbm,
                 acc_sh, idx_s, idx_d, zb_v, *ring):
    rows = ring[:M]
    gsem = ring[M:2 * M]
    c = lax.axis_index("c")
    s = lax.axis_index("s")
    wid = c * NS + s

    def zfill(i, _):
        r = i // (TD // 16)
        q = (i % (TD // 16)) * 16
        zb_v[r, pl.ds(q, 16)] = jnp.zeros((16,), _F32)
        return 0
    lax.fori_loop(0, ZB * TD // 16, zfill, 0)

    def zacc(i, _):
        pltpu.sync_copy(zb_v, acc_sh.at[pl.ds(s * RPT + i * ZB, ZB)])
        return 0
    lax.fori_loop(0, RPT // ZB, zacc, 0)

    pltpu.sync_copy(srcw_hbm.at[wid], idx_s)
    pltpu.sync_copy(dstw_hbm.at[wid], idx_d)

    pltpu.async_copy(table_hbm.at[idx_s.at[0]], rows[0], gsem[0])

    plsc.subcore_barrier()

    for b in range(M - 2):
        pltpu.async_copy(table_hbm.at[idx_s.at[1 + b]], rows[1 + b],
                         gsem[1 + b])

    def step(r, _):
        for q in range(M):
            j = M * r + q
            jn = jnp.minimum(j + M - 1, NCH - 1)
            pltpu.make_async_copy(table_hbm.at[idx_s.at[0]], rows[q],
                                  gsem[q]).wait()

            @pl.when(j + M - 1 < NCH)
            def _():
                pltpu.async_copy(table_hbm.at[idx_s.at[jn]],
                                 rows[(q + M - 1) % M], gsem[(q + M - 1) % M])
            pltpu.sync_copy(rows[q], acc_sh.at[idx_d.at[j]], add=True)
        return 0
    lax.fori_loop(0, NCH // M, step, 0)
    for q in range(NCH % M):
        j = (NCH // M) * M + q
        pltpu.make_async_copy(table_hbm.at[idx_s.at[0]], rows[q],
                              gsem[q]).wait()
        pltpu.sync_copy(rows[q], acc_sh.at[idx_d.at[j]], add=True)

    plsc.subcore_barrier()

    pltpu.sync_copy(acc_sh.at[pl.ds(s * RPT, RPT)],
                    out_hbm.at[c, pl.ds(s * RPT, RPT)])


_sc_agg = pl.kernel(
    _sc_agg_body,
    out_type=jax.ShapeDtypeStruct((NC, ACC_ROWS, TD), _F32),
    mesh=plsc.VectorSubcoreMesh(core_axis_name="c", subcore_axis_name="s",
                                num_cores=NC, num_subcores=NS),
    scratch_types=[
        pltpu.VMEM_SHARED((ACC_ROWS, TD), _F32),
        pltpu.VMEM((NCH, B), jnp.int32),
        pltpu.VMEM((NCH, B), jnp.int32),
        pltpu.VMEM((ZB, TD), _F32),
    ] + [pltpu.VMEM((B, TD), _F32) for _ in range(M)]
      + [pltpu.SemaphoreType.DMA for _ in range(M)],
    compiler_params=pltpu.CompilerParams(use_tc_tiling_on_sc=False,
                                         disable_bounds_checks=True),
)


def _gelu(v):
    return 0.5 * v * (1.0 + lax.erf(v * np.float32(0.7071067811865476)))


def _dot(a, b):
    return jnp.dot(a, b, preferred_element_type=_F32)


def _prep_body(x_r, inWT_r, inb_r, dga_r, dgb_r, h_r, dinv_r, d2_r, dsq_r):
    deg = jnp.maximum(dga_r[0][:, :1] + dgb_r[0][:, :1], 1.0)
    dinv = lax.rsqrt(deg)
    dinv_r[...] = jnp.broadcast_to(dinv, (BLK, TD))
    d2_r[...] = jnp.broadcast_to(dinv * dinv, (BLK, TD))
    dsq_r[...] = jnp.broadcast_to(jnp.sqrt(deg), (BLK, TD))
    h_r[...] = _dot(x_r[...], inWT_r[...]) + inb_r[...]


def _phi_body(h_r, dinv_r, w1_r, b1_r, w2_r, b2_r, w3_r, b3_r, lt_r, ltb_r,
              exp_r, p_r, u_r, rs_r, c_r, s_r):
    h = h_r[...]
    a = _gelu(_dot(h, w1_r[...]) + b1_r[...])
    a = _gelu(_dot(a, w2_r[...]) + b2_r[...])
    ang = _dot(a, w3_r[...]) + b3_r[...]
    C = _dot(jnp.cos(ang), exp_r[...])
    S = _dot(jnp.sin(ang), exp_r[...])
    hb = C * h + S * _dot(h, p_r[...])
    Hm = _dot(hb, lt_r[...]) + ltb_r[...]
    u = dinv_r[...] * Hm
    u_r[...] = u
    rs_r[...] = u
    c_r[...] = C
    s_r[...] = S


def _step_body(u_r, d2_r, ga_r, gb_r, rs_r, un_r, rsn_r, *, coef):
    un = np.float32(coef) * (u_r[...] - d2_r[...] * (ga_r[0] + gb_r[0]))
    un_r[...] = un
    rsn_r[...] = rs_r[...] + un


def _tail_body(h_r, rs_r, dsq_r, c_r, s_r, p_r, hn_r):
    res = rs_r[...] * dsq_r[...]
    hout = c_r[...] * res - s_r[...] * _dot(res, p_r[...])
    hn_r[...] = h_r[...] + _gelu(hout)


def _out_body(h_r, wT_r, b_r, o_r):
    o_r[...] = _dot(h_r[...], wT_r[...]) + b_r[...]


def _rows(d):
    return pl.BlockSpec((BLK, d), lambda i: (i, 0))


def _aggspec(c):
    return pl.BlockSpec((1, BLK, TD), lambda i, c=c: (c, i, 0))


def _full(r, d):
    return pl.BlockSpec((r, d), lambda i: (0, 0))


def _sds(*shapes):
    return [jax.ShapeDtypeStruct(s, _F32) for s in shapes]


_prep = pl.pallas_call(
    _prep_body, grid=(GRID,),
    in_specs=[_rows(D_IN), _full(D_IN, TD), _full(1, TD), _aggspec(0), _aggspec(1)],
    out_specs=[_rows(TD)] * 4,
    out_shape=_sds((N, TD), (N, TD), (N, TD), (N, TD)),
)

_phi = pl.pallas_call(
    _phi_body, grid=(GRID,),
    in_specs=[_rows(TD), _rows(TD), _full(TD, HID), _full(1, HID),
              _full(HID, HID), _full(1, HID), _full(HID, NB), _full(1, NB),
              _full(TD, TD), _full(1, TD), _full(NB, TD), _full(TD, TD)],
    out_specs=[_rows(TD)] * 4,
    out_shape=_sds((N, TD), (N, TD), (N, TD), (N, TD)),
)


def _make_step(coef):
    return pl.pallas_call(
        functools.partial(_step_body, coef=coef), grid=(GRID,),
        in_specs=[_rows(TD), _rows(TD), _aggspec(0), _aggspec(1), _rows(TD)],
        out_specs=[_rows(TD)] * 2,
        out_shape=_sds((N, TD), (N, TD)),
    )


_steps = [_make_step(-T / k) for k in range(1, K + 1)]

_tail = pl.pallas_call(
    _tail_body, grid=(GRID,),
    in_specs=[_rows(TD)] * 5 + [_full(TD, TD)],
    out_specs=_rows(TD),
    out_shape=jax.ShapeDtypeStruct((N, TD), _F32),
)

_out = pl.pallas_call(
    _out_body, grid=(GRID,),
    in_specs=[_rows(TD), _full(TD, D_OUT), _full(1, D_OUT)],
    out_specs=_rows(D_OUT),
    out_shape=jax.ShapeDtypeStruct((N, D_OUT), _F32),
)


def kernel(x, edge_index, in_W, in_b, phi_W1, phi_b1, phi_W2, phi_b2,
           phi_W3, phi_b3, lt_W, lt_b, out_W, out_b):
    exp_m, p_m = _consts()
    src = edge_index[0]
    dst = edge_index[1]
    pad = EP - E
    src_w = jnp.concatenate([src, jnp.zeros((pad,), jnp.int32)]).reshape(NW, NCH, B)
    dst_w = jnp.concatenate([dst, jnp.full((pad,), N, jnp.int32)]).reshape(NW, NCH, B)

    dagg = _sc_agg(jnp.ones((N, TD), _F32), src_w, dst_w)
    h, dinv, d2, dsq = _prep(x, in_W.T, in_b.reshape(1, TD), dagg, dagg)

    for l in range(NL):
        u, rs, C, S = _phi(h, dinv,
                           phi_W1[l].T, phi_b1[l].reshape(1, HID),
                           phi_W2[l].T, phi_b2[l].reshape(1, HID),
                           phi_W3[l].T, phi_b3[l].reshape(1, NB),
                           lt_W[l].T, lt_b[l].reshape(1, TD),
                           exp_m, p_m)
        for k in range(K):
            agg = _sc_agg(u, src_w, dst_w)
            u, rs = _steps[k](u, d2, agg, agg, rs)
        h = _tail(h, rs, dsq, C, S, p_m)

    return _out(h, out_W.T, out_b.reshape(1, D_OUT))


# speedup vs baseline: 1.1946x; 1.0116x over previous
import functools

import numpy as np
import jax
import jax.numpy as jnp
from jax import lax
from jax.experimental import pallas as pl
from jax.experimental.pallas import tpu as pltpu
from jax.experimental.pallas import tpu_sc as plsc

N = 10000
E = 320000
D_IN = 128
D_OUT = 128
NB = 32
TD = 2 * NB
NL = 4
HID = 128
K = 6
T = 1.0

NC = 2
NS = 16
NW = NC * NS
B = 128
M = 4
ZB = 128
EPW = 10112
NCH = EPW // B
RNDS = NCH // M
EP = EPW * NW
ACC_ROWS = 10240
RPT = ACC_ROWS // NS
WPT = N // NS

BLK = 2000
GRID = N // BLK

_F32 = jnp.float32


def _consts():
    exp = np.zeros((NB, TD), np.float32)
    p = np.zeros((TD, TD), np.float32)
    for j in range(NB):
        exp[j, 2 * j] = 1.0
        exp[j, 2 * j + 1] = 1.0
        p[2 * j + 1, 2 * j] = -1.0
        p[2 * j, 2 * j + 1] = 1.0
    return jnp.asarray(exp), jnp.asarray(p)


def _sc_agg_body(table_hbm, srcw_hbm, dstw_hbm, out_hbm,
                 acc_sh, idx_s, idx_d, zb_v, *ring):
    rows = ring[:M]
    gsem = ring[M:2 * M]
    c = lax.axis_index("c")
    s = lax.axis_index("s")
    wid = c * NS + s

    def zfill(i, _):
        r = i // (TD // 16)
        q = (i % (TD // 16)) * 16
        zb_v[r, pl.ds(q, 16)] = jnp.zeros((16,), _F32)
        return 0
    lax.fori_loop(0, ZB * TD // 16, zfill, 0)

    def zacc(i, _):
        pltpu.sync_copy(zb_v, acc_sh.at[pl.ds(s * RPT + i * ZB, ZB)])
        return 0
    lax.fori_loop(0, RPT // ZB, zacc, 0)

    pltpu.sync_copy(srcw_hbm.at[wid], idx_s)
    pltpu.sync_copy(dstw_hbm.at[wid], idx_d)

    pltpu.async_copy(table_hbm.at[idx_s.at[0]], rows[0], gsem[0])

    plsc.subcore_barrier()

    for b in range(M - 2):
        pltpu.async_copy(table_hbm.at[idx_s.at[1 + b]], rows[1 + b],
                         gsem[1 + b])

    def step(r, _):
        for q in range(M):
            j = M * r + q
            jn = jnp.minimum(j + M - 1, NCH - 1)
            pltpu.make_async_copy(table_hbm.at[idx_s.at[0]], rows[q],
                                  gsem[q]).wait()

            @pl.when(j + M - 1 < NCH)
            def _():
                pltpu.async_copy(table_hbm.at[idx_s.at[jn]],
                                 rows[(q + M - 1) % M], gsem[(q + M - 1) % M])
            pltpu.sync_copy(rows[q], acc_sh.at[idx_d.at[j]], add=True)
        return 0
    lax.fori_loop(0, NCH // M, step, 0)
    for q in range(NCH % M):
        j = (NCH // M) * M + q
        pltpu.make_async_copy(table_hbm.at[idx_s.at[0]], rows[q],
                              gsem[q]).wait()
        pltpu.sync_copy(rows[q], acc_sh.at[idx_d.at[j]], add=True)

    plsc.subcore_barrier()

    pltpu.sync_copy(acc_sh.at[pl.ds(s * RPT, RPT)],
                    out_hbm.at[c, pl.ds(s * RPT, RPT)])


_sc_agg = pl.kernel(
    _sc_agg_body,
    out_type=jax.ShapeDtypeStruct((NC, ACC_ROWS, TD), _F32),
    mesh=plsc.VectorSubcoreMesh(core_axis_name="c", subcore_axis_name="s",
                                num_cores=NC, num_subcores=NS),
    scratch_types=[
        pltpu.VMEM_SHARED((ACC_ROWS, TD), _F32),
        pltpu.VMEM((NCH, B), jnp.int32),
        pltpu.VMEM((NCH, B), jnp.int32),
        pltpu.VMEM((ZB, TD), _F32),
    ] + [pltpu.VMEM((B, TD), _F32) for _ in range(M)]
      + [pltpu.SemaphoreType.DMA for _ in range(M)],
    compiler_params=pltpu.CompilerParams(use_tc_tiling_on_sc=False,
                                         disable_bounds_checks=True),
)


def _gelu(v):
    return 0.5 * v * (1.0 + lax.erf(v * np.float32(0.7071067811865476)))


def _dot(a, b):
    return jnp.dot(a, b, preferred_element_type=_F32)


def _prep_body(x_r, inWT_r, inb_r, dga_r, dgb_r, h_r, dinv_r, d2_r, dsq_r):
    deg = jnp.maximum(dga_r[0][:, :1] + dgb_r[0][:, :1], 1.0)
    dinv = lax.rsqrt(deg)
    dinv_r[...] = jnp.broadcast_to(dinv, (BLK, TD))
    d2_r[...] = jnp.broadcast_to(dinv * dinv, (BLK, TD))
    dsq_r[...] = jnp.broadcast_to(jnp.sqrt(deg), (BLK, TD))
    h_r[...] = _dot(x_r[...], inWT_r[...]) + inb_r[...]


def _phi_body(h_r, dinv_r, w1_r, b1_r, w2_r, b2_r, w3_r, b3_r, lt_r, ltb_r,
              exp_r, p_r, u_r, rs_r, c_r, s_r):
    h = h_r[...]
    a = _gelu(_dot(h, w1_r[...]) + b1_r[...])
    a = _gelu(_dot(a, w2_r[...]) + b2_r[...])
    ang = _dot(a, w3_r[...]) + b3_r[...]
    C = _dot(jnp.cos(ang), exp_r[...])
    S = _dot(jnp.sin(ang), exp_r[...])
    hb = C * h + S * _dot(h, p_r[...])
    Hm = _dot(hb, lt_r[...]) + ltb_r[...]
    u = dinv_r[...] * Hm
    u_r[...] = u
    rs_r[...] = u
    c_r[...] = C
    s_r[...] = S


def _step_body(u_r, d2_r, ga_r, gb_r, rs_r, un_r, rsn_r, *, coef):
    un = np.float32(coef) * (u_r[...] - d2_r[...] * (ga_r[0] + gb_r[0]))
    un_r[...] = un
    rsn_r[...] = rs_r[...] + un


def _tail_body(h_r, rs_r, dsq_r, c_r, s_r, p_r, hn_r):
    res = rs_r[...] * dsq_r[...]
    hout = c_r[...] * res - s_r[...] * _dot(res, p_r[...])
    hn_r[...] = h_r[...] + _gelu(hout)


def _out_body(h_r, wT_r, b_r, o_r):
    o_r[...] = _dot(h_r[...], wT_r[...]) + b_r[...]


def _rows(d):
    return pl.BlockSpec((BLK, d), lambda i: (i, 0))


def _aggspec(c):
    return pl.BlockSpec((1, BLK, TD), lambda i, c=c: (c, i, 0))


def _full(r, d):
    return pl.BlockSpec((r, d), lambda i: (0, 0))


def _sds(*shapes):
    return [jax.ShapeDtypeStruct(s, _F32) for s in shapes]


_prep = pl.pallas_call(
    _prep_body, grid=(GRID,),
    in_specs=[_rows(D_IN), _full(D_IN, TD), _full(1, TD), _aggspec(0), _aggspec(1)],
    out_specs=[_rows(TD)] * 4,
    out_shape=_sds((N, TD), (N, TD), (N, TD), (N, TD)),
)

_phi = pl.pallas_call(
    _phi_body, grid=(GRID,),
    in_specs=[_rows(TD), _rows(TD), _full(TD, HID), _full(1, HID),
              _full(HID, HID), _full(1, HID), _full(HID, NB), _full(1, NB),
              _full(TD, TD), _full(1, TD), _full(NB, TD), _full(TD, TD)],
    out_specs=[_rows(TD)] * 4,
    out_shape=_sds((N, TD), (N, TD), (N, TD), (N, TD)),
)


def _make_step(coef):
    return pl.pallas_call(
        functools.partial(_step_body, coef=coef), grid=(GRID,),
        in_specs=[_rows(TD), _rows(TD), _aggspec(0), _aggspec(1), _rows(TD)],
        out_specs=[_rows(TD)] * 2,
        out_shape=_sds((N, TD), (N, TD)),
    )


_steps = [_make_step(-T / k) for k in range(1, K + 1)]

_tail = pl.pallas_call(
    _tail_body, grid=(GRID,),
    in_specs=[_rows(TD)] * 5 + [_full(TD, TD)],
    out_specs=_rows(TD),
    out_shape=jax.ShapeDtypeStruct((N, TD), _F32),
)

_out = pl.pallas_call(
    _out_body, grid=(GRID,),
    in_specs=[_rows(TD), _full(TD, D_OUT), _full(1, D_OUT)],
    out_specs=_rows(D_OUT),
    out_shape=jax.ShapeDtypeStruct((N, D_OUT), _F32),
)


def kernel(x, edge_index, in_W, in_b, phi_W1, phi_b1, phi_W2, phi_b2,
           phi_W3, phi_b3, lt_W, lt_b, out_W, out_b):
    exp_m, p_m = _consts()
    src = edge_index[0]
    dst = edge_index[1]
    pad = EP - E
    src_w = jnp.concatenate([src, jnp.zeros((pad,), jnp.int32)]).reshape(NW, NCH, B)
    dst_w = jnp.concatenate([dst, jnp.full((pad,), N, jnp.int32)]).reshape(NW, NCH, B)

    dagg = _sc_agg(jnp.ones((N, TD), _F32), src_w, dst_w)
    h, dinv, d2, dsq = _prep(x, in_W.T, in_b.reshape(1, TD), dagg, dagg)

    for l in range(NL):
        u, rs, C, S = _phi(h, dinv,
                           phi_W1[l].T, phi_b1[l].reshape(1, HID),
                           phi_W2[l].T, phi_b2[l].reshape(1, HID),
                           phi_W3[l].T, phi_b3[l].reshape(1, NB),
                           lt_W[l].T, lt_b[l].reshape(1, TD),
                           exp_m, p_m)
        for k in range(K):
            agg = _sc_agg(u, src_w, dst_w)
            u, rs = _steps[k](u, d2, agg, agg, rs)
        h = _tail(h, rs, dsq, C, S, p_m)

    return _out(h, out_W.T, out_b.reshape(1, D_OUT))


# speedup vs baseline: 1.2200x; 1.0213x over previous
import functools

import numpy as np
import jax
import jax.numpy as jnp
from jax import lax
from jax.experimental import pallas as pl
from jax.experimental.pallas import tpu as pltpu
from jax.experimental.pallas import tpu_sc as plsc

N = 10000
E = 320000
D_IN = 128
D_OUT = 128
NB = 32
TD = 2 * NB
NL = 4
HID = 128
K = 6
T = 1.0

NC = 2
NS = 16
NW = NC * NS
B = 128
M = 6
ZB = 128
EPW = 10112
NCH = EPW // B
RNDS = NCH // M
EP = EPW * NW
ACC_ROWS = 10240
RPT = ACC_ROWS // NS
WPT = N // NS

BLK = 2000
GRID = N // BLK

_F32 = jnp.float32


def _consts():
    exp = np.zeros((NB, TD), np.float32)
    p = np.zeros((TD, TD), np.float32)
    for j in range(NB):
        exp[j, 2 * j] = 1.0
        exp[j, 2 * j + 1] = 1.0
        p[2 * j + 1, 2 * j] = -1.0
        p[2 * j, 2 * j + 1] = 1.0
    return jnp.asarray(exp), jnp.asarray(p)


def _sc_agg_body(table_hbm, srcw_hbm, dstw_hbm, out_hbm,
                 acc_sh, idx_s, idx_d, zb_v, *ring):
    rows = ring[:M]
    gsem = ring[M:2 * M]
    c = lax.axis_index("c")
    s = lax.axis_index("s")
    wid = c * NS + s

    def zfill(i, _):
        r = i // (TD // 16)
        q = (i % (TD // 16)) * 16
        zb_v[r, pl.ds(q, 16)] = jnp.zeros((16,), _F32)
        return 0
    lax.fori_loop(0, ZB * TD // 16, zfill, 0)

    def zacc(i, _):
        pltpu.sync_copy(zb_v, acc_sh.at[pl.ds(s * RPT + i * ZB, ZB)])
        return 0
    lax.fori_loop(0, RPT // ZB, zacc, 0)

    pltpu.sync_copy(srcw_hbm.at[wid], idx_s)
    pltpu.sync_copy(dstw_hbm.at[wid], idx_d)

    pltpu.async_copy(table_hbm.at[idx_s.at[0]], rows[0], gsem[0])

    plsc.subcore_barrier()

    for b in range(M - 2):
        pltpu.async_copy(table_hbm.at[idx_s.at[1 + b]], rows[1 + b],
                         gsem[1 + b])

    def step(r, _):
        for q in range(M):
            j = M * r + q
            jn = jnp.minimum(j + M - 1, NCH - 1)
            pltpu.make_async_copy(table_hbm.at[idx_s.at[0]], rows[q],
                                  gsem[q]).wait()

            @pl.when(j + M - 1 < NCH)
            def _():
                pltpu.async_copy(table_hbm.at[idx_s.at[jn]],
                                 rows[(q + M - 1) % M], gsem[(q + M - 1) % M])
            pltpu.sync_copy(rows[q], acc_sh.at[idx_d.at[j]], add=True)
        return 0
    lax.fori_loop(0, NCH // M, step, 0)
    for q in range(NCH % M):
        j = (NCH // M) * M + q
        pltpu.make_async_copy(table_hbm.at[idx_s.at[0]], rows[q],
                              gsem[q]).wait()
        pltpu.sync_copy(rows[q], acc_sh.at[idx_d.at[j]], add=True)

    plsc.subcore_barrier()

    pltpu.sync_copy(acc_sh.at[pl.ds(s * RPT, RPT)],
                    out_hbm.at[c, pl.ds(s * RPT, RPT)])


_sc_agg = pl.kernel(
    _sc_agg_body,
    out_type=jax.ShapeDtypeStruct((NC, ACC_ROWS, TD), _F32),
    mesh=plsc.VectorSubcoreMesh(core_axis_name="c", subcore_axis_name="s",
                                num_cores=NC, num_subcores=NS),
    scratch_types=[
        pltpu.VMEM_SHARED((ACC_ROWS, TD), _F32),
        pltpu.VMEM((NCH, B), jnp.int32),
        pltpu.VMEM((NCH, B), jnp.int32),
        pltpu.VMEM((ZB, TD), _F32),
    ] + [pltpu.VMEM((B, TD), _F32) for _ in range(M)]
      + [pltpu.SemaphoreType.DMA for _ in range(M)],
    compiler_params=pltpu.CompilerParams(use_tc_tiling_on_sc=False,
                                         disable_bounds_checks=True),
)


def _gelu(v):
    return 0.5 * v * (1.0 + lax.erf(v * np.float32(0.7071067811865476)))


def _dot(a, b):
    return jnp.dot(a, b, preferred_element_type=_F32)


def _prep_body(x_r, inWT_r, inb_r, dga_r, dgb_r, h_r, dinv_r, d2_r, dsq_r):
    deg = jnp.maximum(dga_r[0][:, :1] + dgb_r[0][:, :1], 1.0)
    dinv = lax.rsqrt(deg)
    dinv_r[...] = jnp.broadcast_to(dinv, (BLK, TD))
    d2_r[...] = jnp.broadcast_to(dinv * dinv, (BLK, TD))
    dsq_r[...] = jnp.broadcast_to(jnp.sqrt(deg), (BLK, TD))
    h_r[...] = _dot(x_r[...], inWT_r[...]) + inb_r[...]


def _phi_body(h_r, dinv_r, w1_r, b1_r, w2_r, b2_r, w3_r, b3_r, lt_r, ltb_r,
              exp_r, p_r, u_r, rs_r, c_r, s_r):
    h = h_r[...]
    a = _gelu(_dot(h, w1_r[...]) + b1_r[...])
    a = _gelu(_dot(a, w2_r[...]) + b2_r[...])
    ang = _dot(a, w3_r[...]) + b3_r[...]
    C = _dot(jnp.cos(ang), exp_r[...])
    S = _dot(jnp.sin(ang), exp_r[...])
    hb = C * h + S * _dot(h, p_r[...])
    Hm = _dot(hb, lt_r[...]) + ltb_r[...]
    u = dinv_r[...] * Hm
    u_r[...] = u
    rs_r[...] = u
    c_r[...] = C
    s_r[...] = S


def _step_body(u_r, d2_r, ga_r, gb_r, rs_r, un_r, rsn_r, *, coef):
    un = np.float32(coef) * (u_r[...] - d2_r[...] * (ga_r[0] + gb_r[0]))
    un_r[...] = un
    rsn_r[...] = rs_r[...] + un


def _tail_body(h_r, rs_r, dsq_r, c_r, s_r, p_r, hn_r):
    res = rs_r[...] * dsq_r[...]
    hout = c_r[...] * res - s_r[...] * _dot(res, p_r[...])
    hn_r[...] = h_r[...] + _gelu(hout)


def _out_body(h_r, wT_r, b_r, o_r):
    o_r[...] = _dot(h_r[...], wT_r[...]) + b_r[...]


def _rows(d):
    return pl.BlockSpec((BLK, d), lambda i: (i, 0))


def _aggspec(c):
    return pl.BlockSpec((1, BLK, TD), lambda i, c=c: (c, i, 0))


def _full(r, d):
    return pl.BlockSpec((r, d), lambda i: (0, 0))


def _sds(*shapes):
    return [jax.ShapeDtypeStruct(s, _F32) for s in shapes]


_prep = pl.pallas_call(
    _prep_body, grid=(GRID,),
    in_specs=[_rows(D_IN), _full(D_IN, TD), _full(1, TD), _aggspec(0), _aggspec(1)],
    out_specs=[_rows(TD)] * 4,
    out_shape=_sds((N, TD), (N, TD), (N, TD), (N, TD)),
)

_phi = pl.pallas_call(
    _phi_body, grid=(GRID,),
    in_specs=[_rows(TD), _rows(TD), _full(TD, HID), _full(1, HID),
              _full(HID, HID), _full(1, HID), _full(HID, NB), _full(1, NB),
              _full(TD, TD), _full(1, TD), _full(NB, TD), _full(TD, TD)],
    out_specs=[_rows(TD)] * 4,
    out_shape=_sds((N, TD), (N, TD), (N, TD), (N, TD)),
)


def _make_step(coef):
    return pl.pallas_call(
        functools.partial(_step_body, coef=coef), grid=(GRID,),
        in_specs=[_rows(TD), _rows(TD), _aggspec(0), _aggspec(1), _rows(TD)],
        out_specs=[_rows(TD)] * 2,
        out_shape=_sds((N, TD), (N, TD)),
    )


_steps = [_make_step(-T / k) for k in range(1, K + 1)]

_tail = pl.pallas_call(
    _tail_body, grid=(GRID,),
    in_specs=[_rows(TD)] * 5 + [_full(TD, TD)],
    out_specs=_rows(TD),
    out_shape=jax.ShapeDtypeStruct((N, TD), _F32),
)

_out = pl.pallas_call(
    _out_body, grid=(GRID,),
    in_specs=[_rows(TD), _full(TD, D_OUT), _full(1, D_OUT)],
    out_specs=_rows(D_OUT),
    out_shape=jax.ShapeDtypeStruct((N, D_OUT), _F32),
)


def kernel(x, edge_index, in_W, in_b, phi_W1, phi_b1, phi_W2, phi_b2,
           phi_W3, phi_b3, lt_W, lt_b, out_W, out_b):
    exp_m, p_m = _consts()
    src = edge_index[0]
    dst = edge_index[1]
    pad = EP - E
    src_w = jnp.concatenate([src, jnp.zeros((pad,), jnp.int32)]).reshape(NW, NCH, B)
    dst_w = jnp.concatenate([dst, jnp.full((pad,), N, jnp.int32)]).reshape(NW, NCH, B)

    dagg = _sc_agg(jnp.ones((N, TD), _F32), src_w, dst_w)
    h, dinv, d2, dsq = _prep(x, in_W.T, in_b.reshape(1, TD), dagg, dagg)

    for l in range(NL):
        u, rs, C, S = _phi(h, dinv,
                           phi_W1[l].T, phi_b1[l].reshape(1, HID),
                           phi_W2[l].T, phi_b2[l].reshape(1, HID),
                           phi_W3[l].T, phi_b3[l].reshape(1, NB),
                           lt_W[l].T, lt_b[l].reshape(1, TD),
                           exp_m, p_m)
        for k in range(K):
            agg = _sc_agg(u, src_w, dst_w)
            u, rs = _steps[k](u, d2, agg, agg, rs)
        h = _tail(h, rs, dsq, C, S, p_m)

    return _out(h, out_W.T, out_b.reshape(1, D_OUT))
